# baseline (speedup 1.0000x reference)
import dataclasses
import functools

import jax
import jax.numpy as jnp
from jax import lax
from jax.experimental import pallas as pl
from jax.experimental.pallas import tpu as pltpu
from jax.experimental.pallas import tpu_sc as plsc

NUM_USERS = 6000
NUM_ITEMS = 3500
NUM_BRANDS = 500
N_NODES = NUM_USERS + NUM_ITEMS + NUM_BRANDS
D = 128
N_EDGES = 320000
N_LAYERS = 3

NC = 2
NS = 16
NW = NC * NS
CW = 128
CH = 80
EPT = CH * CW
E_PAD = NW * EPT
N_PAD = 10240
HALF = N_PAD // 2
SB = 8

_mesh = plsc.VectorSubcoreMesh(core_axis_name="c", subcore_axis_name="s")

_cp = pltpu.CompilerParams()
if "needs_layout_passes" in pltpu.CompilerParams.__dataclass_fields__:
    _cp = dataclasses.replace(_cp, needs_layout_passes=False)


@functools.partial(
    pl.kernel,
    mesh=_mesh,
    compiler_params=_cp,
    out_type=(
        jax.ShapeDtypeStruct((NC, NW, CH, CW), jnp.int32),
        jax.ShapeDtypeStruct((NC, NW, CH, CW), jnp.int32),
        jax.ShapeDtypeStruct((NC, NW, CH, CW), jnp.float32),
        jax.ShapeDtypeStruct((NC, NW, 16), jnp.int32),
    ),
    scratch_types=[
        pltpu.VMEM((CH, CW), jnp.int32),
        pltpu.VMEM((CH, CW), jnp.int32),
        pltpu.VMEM((CH, CW), jnp.float32),
        pltpu.VMEM((CH, CW), jnp.int32),
        pltpu.VMEM((CH, CW), jnp.int32),
        pltpu.VMEM((CH, CW), jnp.float32),
        pltpu.VMEM((CH, CW), jnp.int32),
        pltpu.VMEM((CH, CW), jnp.int32),
        pltpu.VMEM((CH, CW), jnp.float32),
        pltpu.VMEM((16,), jnp.int32),
    ],
)
def _partition_sc(src_hbm, dst_hbm, val_hbm,
                  ps_hbm, pd_hbm, pv_hbm, cnt_hbm,
                  src_v, dst_v, val_v,
                  ls_v, ld_v, lv_v, hs_v, hd_v, hv_v, cnt_v):
    cid = lax.axis_index("c")
    sid = lax.axis_index("s")
    wid = cid * NS + sid

    pltpu.sync_copy(src_hbm.at[wid], src_v)
    pltpu.sync_copy(dst_hbm.at[wid], dst_v)
    pltpu.sync_copy(val_hbm.at[wid], val_v)

    zi = jnp.zeros((16,), jnp.int32)
    zf = jnp.zeros((16,), jnp.float32)

    @pl.loop(0, CH)
    def _(row):
        for k in range(CW // 16):
            sl = pl.ds(k * 16, 16)
            ls_v[row, sl] = zi
            ld_v[row, sl] = zi
            hs_v[row, sl] = zi
            hd_v[row, sl] = zi
            lv_v[row, sl] = zf
            hv_v[row, sl] = zf

    c127 = jnp.full((16,), 127, jnp.int32)

    def body(row, carry):
        ptr_lo, ptr_hi = carry
        for cg in range(CW // 16):
            sl = pl.ds(cg * 16, 16)
            s = src_v[row, sl]
            d = dst_v[row, sl]
            v = val_v[row, sl]
            m = d < HALF
            ones = jnp.where(m, 1, 0).astype(jnp.int32)
            pc_lo = plsc.cumsum(ones)
            pc_hi = plsc.cumsum(1 - ones)
            n_lo = pc_lo[15]
            pos_lo = jnp.full((16,), ptr_lo, jnp.int32) + pc_lo - 1
            pos_hi = jnp.full((16,), ptr_hi, jnp.int32) + pc_hi - 1
            dl = jnp.where(m, d, d - HALF)
            row_lo = jax.lax.shift_right_logical(pos_lo, 7)
            col_lo = jax.lax.bitwise_and(pos_lo, c127)
            row_hi = jax.lax.shift_right_logical(pos_hi, 7)
            col_hi = jax.lax.bitwise_and(pos_hi, c127)
            nm = jnp.logical_not(m)
            plsc.store_scatter(ls_v, [row_lo, col_lo], s, mask=m)
            plsc.store_scatter(ld_v, [row_lo, col_lo], dl, mask=m)
            plsc.store_scatter(lv_v, [row_lo, col_lo], v, mask=m)
            plsc.store_scatter(hs_v, [row_hi, col_hi], s, mask=nm)
            plsc.store_scatter(hd_v, [row_hi, col_hi], dl, mask=nm)
            plsc.store_scatter(hv_v, [row_hi, col_hi], v, mask=nm)
            ptr_lo = ptr_lo + n_lo
            ptr_hi = ptr_hi + (16 - n_lo)
        return ptr_lo, ptr_hi

    n_lo, n_hi = lax.fori_loop(0, CH, body, (jnp.int32(0), jnp.int32(0)))

    pltpu.sync_copy(ls_v, ps_hbm.at[0, wid])
    pltpu.sync_copy(ld_v, pd_hbm.at[0, wid])
    pltpu.sync_copy(lv_v, pv_hbm.at[0, wid])
    pltpu.sync_copy(hs_v, ps_hbm.at[1, wid])
    pltpu.sync_copy(hd_v, pd_hbm.at[1, wid])
    pltpu.sync_copy(hv_v, pv_hbm.at[1, wid])

    cnt_v[...] = jnp.full((16,), n_lo, jnp.int32)
    pltpu.sync_copy(cnt_v, cnt_hbm.at[0, wid])
    cnt_v[...] = jnp.full((16,), n_hi, jnp.int32)
    pltpu.sync_copy(cnt_v, cnt_hbm.at[1, wid])


@functools.partial(
    pl.kernel,
    mesh=_mesh,
    compiler_params=_cp,
    out_type=jax.ShapeDtypeStruct((N_PAD, D), jnp.float32),
    scratch_types=[
        pltpu.VMEM((2 * SB, CW), jnp.int32),
        pltpu.VMEM((2 * SB, CW), jnp.int32),
        pltpu.VMEM((2 * SB, CW), jnp.float32),
        pltpu.VMEM((CW, D), jnp.float32),
        pltpu.VMEM((CW, D), jnp.float32),
        pltpu.VMEM((CW, D), jnp.float32),
        pltpu.VMEM((CW, D), jnp.float32),
        pltpu.VMEM((2, 16), jnp.int32),
        pltpu.VMEM_SHARED((HALF, D), jnp.float32),
        pltpu.SemaphoreType.DMA,
        pltpu.SemaphoreType.DMA,
        pltpu.SemaphoreType.DMA,
        pltpu.SemaphoreType.DMA,
        pltpu.SemaphoreType.DMA,
        pltpu.SemaphoreType.DMA,
        pltpu.SemaphoreType.DMA,
        pltpu.SemaphoreType.DMA,
        pltpu.SemaphoreType.DMA,
    ],
)
def _layer_sc(ego_hbm, ps_hbm, pd_hbm, pv_hbm, cnt_hbm, out_hbm,
              src_v, dst_v, val_v, rb0, rb1, rb2, rb3, len_v, acc_sh,
              g0, g1, g2, g3, s0, s1, s2, s3, sl_sem):
    cid = lax.axis_index("c")
    sid = lax.axis_index("s")
    bufs = (rb0, rb1, rb2, rb3)
    gsems = (g0, g1, g2, g3)
    ssems = (s0, s1, s2, s3)

    pltpu.sync_copy(cnt_hbm.at[cid, 2 * sid], len_v.at[0])
    pltpu.sync_copy(cnt_hbm.at[cid, 2 * sid + 1], len_v.at[1])
    na = len_v[0, pl.ds(0, 16)][0]
    nb = len_v[1, pl.ds(0, 16)][0]
    blk_a = jax.lax.shift_right_logical(na + (SB * CW - 1), 10)
    blk_b = jax.lax.shift_right_logical(nb + (SB * CW - 1), 10)
    tot_b = blk_a + blk_b
    tot_c = tot_b * SB

    def stage(blk):
        sel = jnp.where(blk >= blk_a, 1, 0)
        lst = 2 * sid + sel
        lblk = blk - blk_a * sel
        hoff = pl.multiple_of(lblk * SB, SB)
        voff = pl.multiple_of(jax.lax.bitwise_and(blk, 1) * SB, SB)
        return [
            pltpu.make_async_copy(h.at[cid, lst, pl.ds(hoff, SB)],
                                  v.at[pl.ds(voff, SB)], sl_sem)
            for h, v in ((ps_hbm, src_v), (pd_hbm, dst_v), (pv_hbm, val_v))
        ]

    def stage_start(blk):
        for c in stage(blk):
            c.start()

    def stage_wait(blk):
        for c in stage(blk):
            c.wait()

    def srow_of(c):
        return jax.lax.bitwise_and(c, 2 * SB - 1)

    def gather(c, q):
        return pltpu.make_async_copy(
            ego_hbm.at[src_v.at[srow_of(c)]], bufs[q], gsems[q])

    def scat_start(c, q):
        pltpu.async_copy(bufs[q], acc_sh.at[dst_v.at[srow_of(c)]],
                         ssems[q], add=True)

    def scat_wait(q):
        pltpu.make_async_copy(bufs[q], acc_sh.at[dst_v.at[0]],
                              ssems[q]).wait()

    def scale(c, q):
        srow = srow_of(c)
        buf = bufs[q]

        @pl.loop(0, CW // 16)
        def _(g):
            vvec = val_v[srow, pl.ds(g * 16, 16)]
            for l in range(16):
                vv = jnp.full((16,), vvec[l], jnp.float32)
                e = g * 16 + l
                for k in range(D // 16):
                    sl = pl.ds(k * 16, 16)
                    buf[e, sl] = buf[e, sl] * vv

    zero = jnp.zeros((16,), jnp.float32)

    @pl.loop(0, CW)
    def _(r):
        for k in range(D // 16):
            rb3[r, pl.ds(k * 16, 16)] = zero

    stage_start(0)
    stage_wait(0)
    gather(0, 0).start()
    gather(1, 1).start()

    rows_per_tile = HALF // NS
    base = sid * rows_per_tile
    pltpu.sync_copy(rb3.at[pl.ds(0, CW)], acc_sh.at[pl.ds(base, CW)])
    pltpu.sync_copy(rb3.at[pl.ds(0, CW)], acc_sh.at[pl.ds(base + CW, CW)])
    pltpu.sync_copy(rb3.at[pl.ds(0, 64)], acc_sh.at[pl.ds(base + 2 * CW, 64)])
    plsc.subcore_barrier()

    @pl.loop(0, tot_b)
    def _(blk):
        @pl.when(blk + 1 < tot_b)
        def _():
            stage_start(blk + 1)

        @pl.loop(0, SB, step=4)
        def _(r):
            for i in range(4):
                c = blk * SB + r + i
                gather(c, i).wait()
                scale(c, i)
                scat_start(c, i)
                q2 = (i + 2) % 4
                if i < 2:
                    @pl.when(c >= 2)
                    def _():
                        scat_wait(q2)
                else:
                    scat_wait(q2)

                @pl.when(r + i + 2 == SB)
                def _():
                    @pl.when(blk + 1 < tot_b)
                    def _():
                        stage_wait(blk + 1)

                @pl.when(c + 2 < tot_c)
                def _():
                    gather(c + 2, q2).start()

    scat_wait(2)
    scat_wait(3)
    plsc.subcore_barrier()

    obase = cid * HALF + base
    pltpu.sync_copy(acc_sh.at[pl.ds(base, CW)],
                    out_hbm.at[pl.ds(obase, CW)])
    pltpu.sync_copy(acc_sh.at[pl.ds(base + CW, CW)],
                    out_hbm.at[pl.ds(obase + CW, CW)])
    pltpu.sync_copy(acc_sh.at[pl.ds(base + 2 * CW, 64)],
                    out_hbm.at[pl.ds(obase + 2 * CW, 64)])


def _final_body(e0_ref, e1_ref, e2_ref, e3_ref, o_ref):
    o_ref[...] = 0.25 * (e0_ref[...] + e1_ref[...] + e2_ref[...]
                         + e3_ref[...])


_final = pl.pallas_call(
    _final_body,
    out_shape=jax.ShapeDtypeStruct((N_PAD, D), jnp.float32),
)


@jax.jit
def kernel(edge_index, edge_values, user_emb, item_emb, brand_emb):
    ego0 = jnp.concatenate(
        [user_emb, item_emb, brand_emb,
         jnp.zeros((N_PAD - N_NODES, D), jnp.float32)], axis=0)
    dst = edge_index[0]
    src = edge_index[1]

    pad = E_PAD - N_EDGES
    fill = (jnp.arange(pad, dtype=jnp.int32) * 97) % N_NODES
    src_p = jnp.concatenate([src, fill]).reshape(NW, CH, CW)
    dst_p = jnp.concatenate([dst, fill]).reshape(NW, CH, CW)
    val_p = jnp.concatenate(
        [edge_values, jnp.zeros((pad,), jnp.float32)]).reshape(NW, CH, CW)

    ps, pd, pv, cnt = _partition_sc(src_p, dst_p, val_p)
    ego1 = _layer_sc(ego0, ps, pd, pv, cnt)
    ego2 = _layer_sc(ego1, ps, pd, pv, cnt)
    ego3 = _layer_sc(ego2, ps, pd, pv, cnt)
    final = _final(ego0, ego1, ego2, ego3)

    final_user = final[:NUM_USERS]
    final_item = final[NUM_USERS:NUM_USERS + NUM_ITEMS]
    return (final_user, final_item, user_emb, item_emb)


# speedup vs baseline: 8.4119x; 8.4119x over previous
import dataclasses
import functools

import jax
import jax.numpy as jnp
from jax import lax
from jax.experimental import pallas as pl
from jax.experimental.pallas import tpu as pltpu
from jax.experimental.pallas import tpu_sc as plsc

NUM_USERS = 6000
NUM_ITEMS = 3500
NUM_BRANDS = 500
N_NODES = NUM_USERS + NUM_ITEMS + NUM_BRANDS
D = 128
N_EDGES = 320000
N_LAYERS = 3

NC = 2
NS = 16
NW = NC * NS
CW = 128
CH = 80
EPT = CH * CW
E_PAD = NW * EPT
N_PAD = 10240
HALF = N_PAD // 2
SB = 8

_mesh = plsc.VectorSubcoreMesh(core_axis_name="c", subcore_axis_name="s")

_cp = pltpu.CompilerParams()
if "needs_layout_passes" in pltpu.CompilerParams.__dataclass_fields__:
    _cp = dataclasses.replace(_cp, needs_layout_passes=False)


@functools.partial(
    pl.kernel,
    mesh=_mesh,
    compiler_params=_cp,
    out_type=(
        jax.ShapeDtypeStruct((NC, NW, CH, CW), jnp.int32),
        jax.ShapeDtypeStruct((NC, NW, CH, CW), jnp.int32),
        jax.ShapeDtypeStruct((NC, NW, CH, CW), jnp.float32),
        jax.ShapeDtypeStruct((NC, NW, 16), jnp.int32),
    ),
    scratch_types=[
        pltpu.VMEM((CH, CW), jnp.int32),
        pltpu.VMEM((CH, CW), jnp.int32),
        pltpu.VMEM((CH, CW), jnp.float32),
        pltpu.VMEM((CH, CW), jnp.int32),
        pltpu.VMEM((CH, CW), jnp.int32),
        pltpu.VMEM((CH, CW), jnp.float32),
        pltpu.VMEM((CH, CW), jnp.int32),
        pltpu.VMEM((CH, CW), jnp.int32),
        pltpu.VMEM((CH, CW), jnp.float32),
        pltpu.VMEM((16,), jnp.int32),
    ],
)
def _partition_sc(src_hbm, dst_hbm, val_hbm,
                  ps_hbm, pd_hbm, pv_hbm, cnt_hbm,
                  src_v, dst_v, val_v,
                  ls_v, ld_v, lv_v, hs_v, hd_v, hv_v, cnt_v):
    cid = lax.axis_index("c")
    sid = lax.axis_index("s")
    wid = cid * NS + sid

    pltpu.sync_copy(src_hbm.at[wid], src_v)
    pltpu.sync_copy(dst_hbm.at[wid], dst_v)
    pltpu.sync_copy(val_hbm.at[wid], val_v)

    zf = jnp.zeros((16,), jnp.float32)
    it = jnp.arange(16, dtype=jnp.int32)

    @pl.loop(0, CH)
    def _(row):
        for k in range(CW // 16):
            sl = pl.ds(k * 16, 16)
            mix = row * CW + k * 16
            dspread = jax.lax.bitwise_and(it * 313 + mix,
                                          jnp.full((16,), 4095, jnp.int32))
            sspread = jax.lax.bitwise_and(it * 641 + mix,
                                          jnp.full((16,), 8191, jnp.int32))
            ls_v[row, sl] = sspread
            ld_v[row, sl] = dspread
            hs_v[row, sl] = sspread
            hd_v[row, sl] = dspread
            lv_v[row, sl] = zf
            hv_v[row, sl] = zf

    c127 = jnp.full((16,), 127, jnp.int32)

    def body(row, carry):
        ptr_lo, ptr_hi = carry
        for cg in range(CW // 16):
            sl = pl.ds(cg * 16, 16)
            s = src_v[row, sl]
            d = dst_v[row, sl]
            v = val_v[row, sl]
            m = d < HALF
            ones = jnp.where(m, 1, 0).astype(jnp.int32)
            pc_lo = plsc.cumsum(ones)
            pc_hi = plsc.cumsum(1 - ones)
            n_lo = pc_lo[15]
            pos_lo = jnp.full((16,), ptr_lo, jnp.int32) + pc_lo - 1
            pos_hi = jnp.full((16,), ptr_hi, jnp.int32) + pc_hi - 1
            dl = jnp.where(m, d, d - HALF)
            row_lo = jax.lax.shift_right_logical(pos_lo, 7)
            col_lo = jax.lax.bitwise_and(pos_lo, c127)
            row_hi = jax.lax.shift_right_logical(pos_hi, 7)
            col_hi = jax.lax.bitwise_and(pos_hi, c127)
            nm = jnp.logical_not(m)
            plsc.store_scatter(ls_v, [row_lo, col_lo], s, mask=m)
            plsc.store_scatter(ld_v, [row_lo, col_lo], dl, mask=m)
            plsc.store_scatter(lv_v, [row_lo, col_lo], v, mask=m)
            plsc.store_scatter(hs_v, [row_hi, col_hi], s, mask=nm)
            plsc.store_scatter(hd_v, [row_hi, col_hi], dl, mask=nm)
            plsc.store_scatter(hv_v, [row_hi, col_hi], v, mask=nm)
            ptr_lo = ptr_lo + n_lo
            ptr_hi = ptr_hi + (16 - n_lo)
        return ptr_lo, ptr_hi

    n_lo, n_hi = lax.fori_loop(0, CH, body, (jnp.int32(0), jnp.int32(0)))

    pltpu.sync_copy(ls_v, ps_hbm.at[0, wid])
    pltpu.sync_copy(ld_v, pd_hbm.at[0, wid])
    pltpu.sync_copy(lv_v, pv_hbm.at[0, wid])
    pltpu.sync_copy(hs_v, ps_hbm.at[1, wid])
    pltpu.sync_copy(hd_v, pd_hbm.at[1, wid])
    pltpu.sync_copy(hv_v, pv_hbm.at[1, wid])

    cnt_v[...] = jnp.full((16,), n_lo, jnp.int32)
    pltpu.sync_copy(cnt_v, cnt_hbm.at[0, wid])
    cnt_v[...] = jnp.full((16,), n_hi, jnp.int32)
    pltpu.sync_copy(cnt_v, cnt_hbm.at[1, wid])


@functools.partial(
    pl.kernel,
    mesh=_mesh,
    compiler_params=_cp,
    out_type=jax.ShapeDtypeStruct((N_PAD, D), jnp.float32),
    scratch_types=[
        pltpu.VMEM((2 * SB, CW), jnp.int32),
        pltpu.VMEM((2 * SB, CW), jnp.int32),
        pltpu.VMEM((2 * SB, CW), jnp.float32),
        pltpu.VMEM((CW, D), jnp.float32),
        pltpu.VMEM((CW, D), jnp.float32),
        pltpu.VMEM((CW, D), jnp.float32),
        pltpu.VMEM((CW, D), jnp.float32),
        pltpu.VMEM((2, 16), jnp.int32),
        pltpu.VMEM_SHARED((HALF, D), jnp.float32),
        pltpu.SemaphoreType.DMA,
        pltpu.SemaphoreType.DMA,
        pltpu.SemaphoreType.DMA,
        pltpu.SemaphoreType.DMA,
        pltpu.SemaphoreType.DMA,
        pltpu.SemaphoreType.DMA,
        pltpu.SemaphoreType.DMA,
        pltpu.SemaphoreType.DMA,
        pltpu.SemaphoreType.DMA,
    ],
)
def _layer_sc(ego_hbm, ps_hbm, pd_hbm, pv_hbm, cnt_hbm, out_hbm,
              src_v, dst_v, val_v, rb0, rb1, rb2, rb3, len_v, acc_sh,
              g0, g1, g2, g3, s0, s1, s2, s3, sl_sem):
    cid = lax.axis_index("c")
    sid = lax.axis_index("s")
    bufs = (rb0, rb1, rb2, rb3)
    gsems = (g0, g1, g2, g3)
    ssems = (s0, s1, s2, s3)

    pltpu.sync_copy(cnt_hbm.at[cid, 2 * sid], len_v.at[0])
    pltpu.sync_copy(cnt_hbm.at[cid, 2 * sid + 1], len_v.at[1])
    na = len_v[0, pl.ds(0, 16)][0]
    nb = len_v[1, pl.ds(0, 16)][0]
    blk_a = jax.lax.shift_right_logical(na + (SB * CW - 1), 10)
    blk_b = jax.lax.shift_right_logical(nb + (SB * CW - 1), 10)
    tot_b = blk_a + blk_b
    tot_c = tot_b * SB

    def stage(blk):
        sel = jnp.where(blk >= blk_a, 1, 0)
        lst = 2 * sid + sel
        lblk = blk - blk_a * sel
        hoff = pl.multiple_of(lblk * SB, SB)
        voff = pl.multiple_of(jax.lax.bitwise_and(blk, 1) * SB, SB)
        return [
            pltpu.make_async_copy(h.at[cid, lst, pl.ds(hoff, SB)],
                                  v.at[pl.ds(voff, SB)], sl_sem)
            for h, v in ((ps_hbm, src_v), (pd_hbm, dst_v), (pv_hbm, val_v))
        ]

    def stage_start(blk):
        for c in stage(blk):
            c.start()

    def stage_wait(blk):
        for c in stage(blk):
            c.wait()

    def srow_of(c):
        return jax.lax.bitwise_and(c, 2 * SB - 1)

    def gather(c, q):
        return pltpu.make_async_copy(
            ego_hbm.at[src_v.at[srow_of(c)]], bufs[q], gsems[q])

    def scat_start(c, q):
        pltpu.async_copy(bufs[q], acc_sh.at[dst_v.at[srow_of(c)]],
                         ssems[q], add=True)

    def scat_wait(q):
        pltpu.make_async_copy(bufs[q], acc_sh.at[dst_v.at[0]],
                              ssems[q]).wait()

    def scale(c, q):
        srow = srow_of(c)
        buf = bufs[q]

        @pl.loop(0, CW // 16)
        def _(g):
            vvec = val_v[srow, pl.ds(g * 16, 16)]
            for l in range(16):
                vv = jnp.full((16,), vvec[l], jnp.float32)
                e = g * 16 + l
                for k in range(D // 16):
                    sl = pl.ds(k * 16, 16)
                    buf[e, sl] = buf[e, sl] * vv

    zero = jnp.zeros((16,), jnp.float32)

    @pl.loop(0, CW)
    def _(r):
        for k in range(D // 16):
            rb3[r, pl.ds(k * 16, 16)] = zero

    stage_start(0)
    stage_wait(0)
    gather(0, 0).start()
    gather(1, 1).start()

    rows_per_tile = HALF // NS
    base = sid * rows_per_tile
    pltpu.sync_copy(rb3.at[pl.ds(0, CW)], acc_sh.at[pl.ds(base, CW)])
    pltpu.sync_copy(rb3.at[pl.ds(0, CW)], acc_sh.at[pl.ds(base + CW, CW)])
    pltpu.sync_copy(rb3.at[pl.ds(0, 64)], acc_sh.at[pl.ds(base + 2 * CW, 64)])
    plsc.subcore_barrier()

    @pl.loop(0, tot_b)
    def _(blk):
        @pl.when(blk + 1 < tot_b)
        def _():
            stage_start(blk + 1)

        @pl.loop(0, SB, step=4)
        def _(r):
            for i in range(4):
                c = blk * SB + r + i
                gather(c, i).wait()
                scale(c, i)
                scat_start(c, i)
                q2 = (i + 2) % 4
                if i < 2:
                    @pl.when(c >= 2)
                    def _():
                        scat_wait(q2)
                else:
                    scat_wait(q2)

                @pl.when(r + i + 2 == SB)
                def _():
                    @pl.when(blk + 1 < tot_b)
                    def _():
                        stage_wait(blk + 1)

                @pl.when(c + 2 < tot_c)
                def _():
                    gather(c + 2, q2).start()

    scat_wait(2)
    scat_wait(3)
    plsc.subcore_barrier()

    obase = cid * HALF + base
    pltpu.sync_copy(acc_sh.at[pl.ds(base, CW)],
                    out_hbm.at[pl.ds(obase, CW)])
    pltpu.sync_copy(acc_sh.at[pl.ds(base + CW, CW)],
                    out_hbm.at[pl.ds(obase + CW, CW)])
    pltpu.sync_copy(acc_sh.at[pl.ds(base + 2 * CW, 64)],
                    out_hbm.at[pl.ds(obase + 2 * CW, 64)])


def _final_body(e0_ref, e1_ref, e2_ref, e3_ref, o_ref):
    o_ref[...] = 0.25 * (e0_ref[...] + e1_ref[...] + e2_ref[...]
                         + e3_ref[...])


_final = pl.pallas_call(
    _final_body,
    out_shape=jax.ShapeDtypeStruct((N_PAD, D), jnp.float32),
)


@jax.jit
def kernel(edge_index, edge_values, user_emb, item_emb, brand_emb):
    ego0 = jnp.concatenate(
        [user_emb, item_emb, brand_emb,
         jnp.zeros((N_PAD - N_NODES, D), jnp.float32)], axis=0)
    dst = edge_index[0]
    src = edge_index[1]

    pad = E_PAD - N_EDGES
    fill = (jnp.arange(pad, dtype=jnp.int32) * 97) % N_NODES
    src_p = jnp.concatenate([src, fill]).reshape(NW, CH, CW)
    dst_p = jnp.concatenate([dst, fill]).reshape(NW, CH, CW)
    val_p = jnp.concatenate(
        [edge_values, jnp.zeros((pad,), jnp.float32)]).reshape(NW, CH, CW)

    ps, pd, pv, cnt = _partition_sc(src_p, dst_p, val_p)
    ego1 = _layer_sc(ego0, ps, pd, pv, cnt)
    ego2 = _layer_sc(ego1, ps, pd, pv, cnt)
    ego3 = _layer_sc(ego2, ps, pd, pv, cnt)
    final = _final(ego0, ego1, ego2, ego3)

    final_user = final[:NUM_USERS]
    final_item = final[NUM_USERS:NUM_USERS + NUM_ITEMS]
    return (final_user, final_item, user_emb, item_emb)


# speedup vs baseline: 8.6043x; 1.0229x over previous
import dataclasses
import functools

import jax
import jax.numpy as jnp
from jax import lax
from jax.experimental import pallas as pl
from jax.experimental.pallas import tpu as pltpu
from jax.experimental.pallas import tpu_sc as plsc

NUM_USERS = 6000
NUM_ITEMS = 3500
NUM_BRANDS = 500
N_NODES = NUM_USERS + NUM_ITEMS + NUM_BRANDS
D = 128
N_EDGES = 320000
N_LAYERS = 3

NC = 2
NS = 16
NW = NC * NS
CW = 128
CH = 80
EPT = CH * CW
E_PAD = NW * EPT
N_PAD = 10240
HALF = 4992
ACC_ROWS = 5248
SB = 8

_mesh = plsc.VectorSubcoreMesh(core_axis_name="c", subcore_axis_name="s")

_cp = pltpu.CompilerParams()
if "needs_layout_passes" in pltpu.CompilerParams.__dataclass_fields__:
    _cp = dataclasses.replace(_cp, needs_layout_passes=False)


@functools.partial(
    pl.kernel,
    mesh=_mesh,
    compiler_params=_cp,
    out_type=(
        jax.ShapeDtypeStruct((NC, NW, CH, CW), jnp.int32),
        jax.ShapeDtypeStruct((NC, NW, CH, CW), jnp.int32),
        jax.ShapeDtypeStruct((NC, NW, CH, CW), jnp.float32),
        jax.ShapeDtypeStruct((NC, NW, 16), jnp.int32),
    ),
    scratch_types=[
        pltpu.VMEM((CH, CW), jnp.int32),
        pltpu.VMEM((CH, CW), jnp.int32),
        pltpu.VMEM((CH, CW), jnp.float32),
        pltpu.VMEM((CH, CW), jnp.int32),
        pltpu.VMEM((CH, CW), jnp.int32),
        pltpu.VMEM((CH, CW), jnp.float32),
        pltpu.VMEM((CH, CW), jnp.int32),
        pltpu.VMEM((CH, CW), jnp.int32),
        pltpu.VMEM((CH, CW), jnp.float32),
        pltpu.VMEM((16,), jnp.int32),
    ],
)
def _partition_sc(src_hbm, dst_hbm, val_hbm,
                  ps_hbm, pd_hbm, pv_hbm, cnt_hbm,
                  src_v, dst_v, val_v,
                  ls_v, ld_v, lv_v, hs_v, hd_v, hv_v, cnt_v):
    cid = lax.axis_index("c")
    sid = lax.axis_index("s")
    wid = cid * NS + sid

    pltpu.sync_copy(src_hbm.at[wid], src_v)
    pltpu.sync_copy(dst_hbm.at[wid], dst_v)
    pltpu.sync_copy(val_hbm.at[wid], val_v)

    zf = jnp.zeros((16,), jnp.float32)
    it = jnp.arange(16, dtype=jnp.int32)

    @pl.loop(0, CH)
    def _(row):
        for k in range(CW // 16):
            sl = pl.ds(k * 16, 16)
            mix = row * CW + k * 16
            dspread = jax.lax.bitwise_and(it * 313 + mix,
                                          jnp.full((16,), 4095, jnp.int32))
            sspread = jax.lax.bitwise_and(it * 641 + mix,
                                          jnp.full((16,), 8191, jnp.int32))
            ls_v[row, sl] = sspread
            ld_v[row, sl] = dspread
            hs_v[row, sl] = sspread
            hd_v[row, sl] = dspread
            lv_v[row, sl] = zf
            hv_v[row, sl] = zf

    c127 = jnp.full((16,), 127, jnp.int32)

    def body(row, carry):
        ptr_lo, ptr_hi = carry
        for cg in range(CW // 16):
            sl = pl.ds(cg * 16, 16)
            s = src_v[row, sl]
            d = dst_v[row, sl]
            v = val_v[row, sl]
            m = d < HALF
            ones = jnp.where(m, 1, 0).astype(jnp.int32)
            pc_lo = plsc.cumsum(ones)
            pc_hi = plsc.cumsum(1 - ones)
            n_lo = pc_lo[15]
            pos_lo = jnp.full((16,), ptr_lo, jnp.int32) + pc_lo - 1
            pos_hi = jnp.full((16,), ptr_hi, jnp.int32) + pc_hi - 1
            dl = jnp.where(m, d, d - HALF)
            row_lo = jax.lax.shift_right_logical(pos_lo, 7)
            col_lo = jax.lax.bitwise_and(pos_lo, c127)
            row_hi = jax.lax.shift_right_logical(pos_hi, 7)
            col_hi = jax.lax.bitwise_and(pos_hi, c127)
            nm = jnp.logical_not(m)
            plsc.store_scatter(ls_v, [row_lo, col_lo], s, mask=m)
            plsc.store_scatter(ld_v, [row_lo, col_lo], dl, mask=m)
            plsc.store_scatter(lv_v, [row_lo, col_lo], v, mask=m)
            plsc.store_scatter(hs_v, [row_hi, col_hi], s, mask=nm)
            plsc.store_scatter(hd_v, [row_hi, col_hi], dl, mask=nm)
            plsc.store_scatter(hv_v, [row_hi, col_hi], v, mask=nm)
            ptr_lo = ptr_lo + n_lo
            ptr_hi = ptr_hi + (16 - n_lo)
        return ptr_lo, ptr_hi

    n_lo, n_hi = lax.fori_loop(0, CH, body, (jnp.int32(0), jnp.int32(0)))

    pltpu.sync_copy(ls_v, ps_hbm.at[0, wid])
    pltpu.sync_copy(ld_v, pd_hbm.at[0, wid])
    pltpu.sync_copy(lv_v, pv_hbm.at[0, wid])
    pltpu.sync_copy(hs_v, ps_hbm.at[1, wid])
    pltpu.sync_copy(hd_v, pd_hbm.at[1, wid])
    pltpu.sync_copy(hv_v, pv_hbm.at[1, wid])

    cnt_v[...] = jnp.full((16,), n_lo, jnp.int32)
    pltpu.sync_copy(cnt_v, cnt_hbm.at[0, wid])
    cnt_v[...] = jnp.full((16,), n_hi, jnp.int32)
    pltpu.sync_copy(cnt_v, cnt_hbm.at[1, wid])


@functools.partial(
    pl.kernel,
    mesh=_mesh,
    compiler_params=_cp,
    out_type=jax.ShapeDtypeStruct((N_PAD, D), jnp.float32),
    scratch_types=[
        pltpu.VMEM((2 * SB, CW), jnp.int32),
        pltpu.VMEM((2 * SB, CW), jnp.int32),
        pltpu.VMEM((2 * SB, CW), jnp.float32),
        pltpu.VMEM((CW, D), jnp.float32),
        pltpu.VMEM((CW, D), jnp.float32),
        pltpu.VMEM((CW, D), jnp.float32),
        pltpu.VMEM((CW, D), jnp.float32),
        pltpu.VMEM((2, 16), jnp.int32),
        pltpu.VMEM_SHARED((ACC_ROWS, D), jnp.float32),
        pltpu.SemaphoreType.DMA,
        pltpu.SemaphoreType.DMA,
        pltpu.SemaphoreType.DMA,
        pltpu.SemaphoreType.DMA,
        pltpu.SemaphoreType.DMA,
        pltpu.SemaphoreType.DMA,
        pltpu.SemaphoreType.DMA,
        pltpu.SemaphoreType.DMA,
        pltpu.SemaphoreType.DMA,
    ],
)
def _layer_sc(ego_hbm, ps_hbm, pd_hbm, pv_hbm, cnt_hbm, out_hbm,
              src_v, dst_v, val_v, rb0, rb1, rb2, rb3, len_v, acc_sh,
              g0, g1, g2, g3, s0, s1, s2, s3, sl_sem):
    cid = lax.axis_index("c")
    sid = lax.axis_index("s")
    bufs = (rb0, rb1, rb2, rb3)
    gsems = (g0, g1, g2, g3)
    ssems = (s0, s1, s2, s3)

    pltpu.sync_copy(cnt_hbm.at[cid, 2 * sid], len_v.at[0])
    pltpu.sync_copy(cnt_hbm.at[cid, 2 * sid + 1], len_v.at[1])
    na = len_v[0, pl.ds(0, 16)][0]
    nb = len_v[1, pl.ds(0, 16)][0]
    blk_a = jax.lax.shift_right_logical(na + (SB * CW - 1), 10)
    blk_b = jax.lax.shift_right_logical(nb + (SB * CW - 1), 10)
    tot_b = blk_a + blk_b
    tot_c = tot_b * SB

    def stage(blk):
        sel = jnp.where(blk >= blk_a, 1, 0)
        lst = 2 * sid + sel
        lblk = blk - blk_a * sel
        hoff = pl.multiple_of(lblk * SB, SB)
        voff = pl.multiple_of(jax.lax.bitwise_and(blk, 1) * SB, SB)
        return [
            pltpu.make_async_copy(h.at[cid, lst, pl.ds(hoff, SB)],
                                  v.at[pl.ds(voff, SB)], sl_sem)
            for h, v in ((ps_hbm, src_v), (pd_hbm, dst_v), (pv_hbm, val_v))
        ]

    def stage_start(blk):
        for c in stage(blk):
            c.start()

    def stage_wait(blk):
        for c in stage(blk):
            c.wait()

    def srow_of(c):
        return jax.lax.bitwise_and(c, 2 * SB - 1)

    def gather(c, q):
        return pltpu.make_async_copy(
            ego_hbm.at[src_v.at[srow_of(c)]], bufs[q], gsems[q])

    def scat_start(c, q):
        pltpu.async_copy(bufs[q], acc_sh.at[dst_v.at[srow_of(c)]],
                         ssems[q], add=True)

    def scat_wait(q):
        pltpu.make_async_copy(bufs[q], acc_sh.at[dst_v.at[0]],
                              ssems[q]).wait()

    def scale(c, q):
        srow = srow_of(c)
        buf = bufs[q]

        @pl.loop(0, CW // 16)
        def _(g):
            vvec = val_v[srow, pl.ds(g * 16, 16)]
            for l in range(16):
                vv = jnp.full((16,), vvec[l], jnp.float32)
                e = g * 16 + l
                for k in range(D // 16):
                    sl = pl.ds(k * 16, 16)
                    buf[e, sl] = buf[e, sl] * vv

    zero = jnp.zeros((16,), jnp.float32)

    @pl.loop(0, CW)
    def _(r):
        for k in range(D // 16):
            rb3[r, pl.ds(k * 16, 16)] = zero

    stage_start(0)
    stage_wait(0)
    gather(0, 0).start()
    gather(1, 1).start()

    @pl.when(cid == 0)
    def _():
        base = sid * 312
        pltpu.sync_copy(rb3.at[pl.ds(0, CW)], acc_sh.at[pl.ds(base, CW)])
        pltpu.sync_copy(rb3.at[pl.ds(0, CW)], acc_sh.at[pl.ds(base + CW, CW)])
        pltpu.sync_copy(rb3.at[pl.ds(0, 56)],
                        acc_sh.at[pl.ds(base + 2 * CW, 56)])

    @pl.when(cid == 1)
    def _():
        base = sid * 328
        pltpu.sync_copy(rb3.at[pl.ds(0, CW)], acc_sh.at[pl.ds(base, CW)])
        pltpu.sync_copy(rb3.at[pl.ds(0, CW)], acc_sh.at[pl.ds(base + CW, CW)])
        pltpu.sync_copy(rb3.at[pl.ds(0, 72)],
                        acc_sh.at[pl.ds(base + 2 * CW, 72)])

    plsc.subcore_barrier()

    @pl.loop(0, tot_b)
    def _(blk):
        @pl.when(blk + 1 < tot_b)
        def _():
            stage_start(blk + 1)

        @pl.loop(0, SB, step=4)
        def _(r):
            for i in range(4):
                c = blk * SB + r + i
                gather(c, i).wait()
                scale(c, i)
                scat_start(c, i)
                q2 = (i + 2) % 4
                if i < 2:
                    @pl.when(c >= 2)
                    def _():
                        scat_wait(q2)
                else:
                    scat_wait(q2)

                @pl.when(r + i + 2 == SB)
                def _():
                    @pl.when(blk + 1 < tot_b)
                    def _():
                        stage_wait(blk + 1)

                @pl.when(c + 2 < tot_c)
                def _():
                    gather(c + 2, q2).start()

    scat_wait(2)
    scat_wait(3)
    plsc.subcore_barrier()

    @pl.when(cid == 0)
    def _():
        base = sid * 312
        pltpu.sync_copy(acc_sh.at[pl.ds(base, CW)],
                        out_hbm.at[pl.ds(base, CW)])
        pltpu.sync_copy(acc_sh.at[pl.ds(base + CW, CW)],
                        out_hbm.at[pl.ds(base + CW, CW)])
        pltpu.sync_copy(acc_sh.at[pl.ds(base + 2 * CW, 56)],
                        out_hbm.at[pl.ds(base + 2 * CW, 56)])

    @pl.when(cid == 1)
    def _():
        base = sid * 328
        obase = HALF + base
        pltpu.sync_copy(acc_sh.at[pl.ds(base, CW)],
                        out_hbm.at[pl.ds(obase, CW)])
        pltpu.sync_copy(acc_sh.at[pl.ds(base + CW, CW)],
                        out_hbm.at[pl.ds(obase + CW, CW)])
        pltpu.sync_copy(acc_sh.at[pl.ds(base + 2 * CW, 72)],
                        out_hbm.at[pl.ds(obase + 2 * CW, 72)])


def _final_body(e0_ref, e1_ref, e2_ref, e3_ref, o_ref):
    o_ref[...] = 0.25 * (e0_ref[...] + e1_ref[...] + e2_ref[...]
                         + e3_ref[...])


_final = pl.pallas_call(
    _final_body,
    out_shape=jax.ShapeDtypeStruct((N_PAD, D), jnp.float32),
)


@jax.jit
def kernel(edge_index, edge_values, user_emb, item_emb, brand_emb):
    ego0 = jnp.concatenate(
        [user_emb, item_emb, brand_emb,
         jnp.zeros((N_PAD - N_NODES, D), jnp.float32)], axis=0)
    dst = edge_index[0]
    src = edge_index[1]

    pad = E_PAD - N_EDGES
    fill = (jnp.arange(pad, dtype=jnp.int32) * 97) % N_NODES
    src_p = jnp.concatenate([src, fill]).reshape(NW, CH, CW)
    dst_p = jnp.concatenate([dst, fill]).reshape(NW, CH, CW)
    val_p = jnp.concatenate(
        [edge_values, jnp.zeros((pad,), jnp.float32)]).reshape(NW, CH, CW)

    ps, pd, pv, cnt = _partition_sc(src_p, dst_p, val_p)
    ego1 = _layer_sc(ego0, ps, pd, pv, cnt)
    ego2 = _layer_sc(ego1, ps, pd, pv, cnt)
    ego3 = _layer_sc(ego2, ps, pd, pv, cnt)
    final = _final(ego0, ego1, ego2, ego3)

    final_user = final[:NUM_USERS]
    final_item = final[NUM_USERS:NUM_USERS + NUM_ITEMS]
    return (final_user, final_item, user_emb, item_emb)


# speedup vs baseline: 8.7858x; 1.0211x over previous
import dataclasses
import functools

import jax
import jax.numpy as jnp
from jax import lax
from jax.experimental import pallas as pl
from jax.experimental.pallas import tpu as pltpu
from jax.experimental.pallas import tpu_sc as plsc

NUM_USERS = 6000
NUM_ITEMS = 3500
NUM_BRANDS = 500
N_NODES = NUM_USERS + NUM_ITEMS + NUM_BRANDS
D = 128
N_EDGES = 320000
N_LAYERS = 3

NC = 2
NS = 16
NW = NC * NS
CW = 128
CH = 80
EPT = CH * CW
E_PAD = NW * EPT
N_PAD = 10240
ROWS_PER_TILE = N_PAD // NS
WB = 128
SB = 8
NB = CH // SB

_mesh = plsc.VectorSubcoreMesh(core_axis_name="c", subcore_axis_name="s")

_cp = pltpu.CompilerParams()
if "needs_layout_passes" in pltpu.CompilerParams.__dataclass_fields__:
    _cp = dataclasses.replace(_cp, needs_layout_passes=False)


@functools.partial(
    pl.kernel,
    mesh=_mesh,
    compiler_params=_cp,
    out_type=jax.ShapeDtypeStruct((NC, N_PAD, D), jnp.float32),
    scratch_types=[
        pltpu.VMEM((2 * SB, CW), jnp.int32),
        pltpu.VMEM((2 * SB, CW), jnp.int32),
        pltpu.VMEM((2 * SB, CW), jnp.float32),
        pltpu.VMEM((CW, D), jnp.float32),
        pltpu.VMEM((CW, D), jnp.float32),
        pltpu.VMEM_SHARED((N_PAD, D), jnp.float32),
        pltpu.SemaphoreType.DMA,
        pltpu.SemaphoreType.DMA,
        pltpu.SemaphoreType.DMA,
    ],
)
def _layer_sc(ego_hbm, src_hbm, dst_hbm, val_hbm, out_hbm,
              src_v, dst_v, val_v, rows_a, rows_b, acc_sh,
              gs0, gs1, sl_sem):
    cid = lax.axis_index("c")
    sid = lax.axis_index("s")
    wid = cid * NS + sid

    def slab_copies(bb, p):
        return [
            pltpu.make_async_copy(h.at[wid, pl.ds(bb * SB, SB)],
                                  v.at[pl.ds(p * SB, SB)], sl_sem)
            for h, v in ((src_hbm, src_v), (dst_hbm, dst_v),
                         (val_hbm, val_v))
        ]

    def stage_start(bb, p):
        for c in slab_copies(bb, p):
            c.start()

    def stage_wait(bb, p):
        for c in slab_copies(bb, p):
            c.wait()

    def gather(srow, buf, sem):
        return pltpu.make_async_copy(ego_hbm.at[src_v.at[srow]], buf, sem)

    def scale(srow, buf):
        @pl.loop(0, CW // 16)
        def _(g):
            vvec = val_v[srow, pl.ds(g * 16, 16)]
            for l in range(16):
                vv = jnp.full((16,), vvec[l], jnp.float32)
                e = g * 16 + l
                for k in range(D // 16):
                    sl = pl.ds(k * 16, 16)
                    buf[e, sl] = buf[e, sl] * vv

    stage_start(0, 0)
    stage_wait(0, 0)

    zero = jnp.zeros((16,), jnp.float32)

    @pl.loop(0, CW)
    def _(r):
        for k in range(D // 16):
            rows_b[r, pl.ds(k * 16, 16)] = zero

    gather(0, rows_a, gs0).start()

    base = sid * ROWS_PER_TILE
    for i in range(ROWS_PER_TILE // WB):
        pltpu.sync_copy(rows_b.at[pl.ds(0, WB)],
                        acc_sh.at[pl.ds(base + i * WB, WB)])
    plsc.subcore_barrier()

    @pl.loop(0, NB)
    def _(bb):
        p = bb % 2

        @pl.when(bb + 1 < NB)
        def _():
            stage_start(bb + 1, 1 - p)

        @pl.loop(0, SB, step=2)
        def _(r):
            srow = p * SB + r
            gather(srow, rows_a, gs0).wait()
            gather(srow + 1, rows_b, gs1).start()
            scale(srow, rows_a)
            pltpu.sync_copy(rows_a, acc_sh.at[dst_v.at[srow]], add=True)
            gather(srow + 1, rows_b, gs1).wait()

            @pl.when(r + 2 < SB)
            def _():
                gather(srow + 2, rows_a, gs0).start()

            @pl.when(r + 2 >= SB)
            def _():
                @pl.when(bb + 1 < NB)
                def _():
                    stage_wait(bb + 1, 1 - p)
                    gather((1 - p) * SB, rows_a, gs0).start()

            scale(srow + 1, rows_b)
            pltpu.sync_copy(rows_b, acc_sh.at[dst_v.at[srow + 1]], add=True)

    plsc.subcore_barrier()

    for i in range(ROWS_PER_TILE // WB):
        pltpu.sync_copy(acc_sh.at[pl.ds(base + i * WB, WB)],
                        out_hbm.at[cid, pl.ds(base + i * WB, WB)])


def _add2_body(p_ref, o_ref):
    o_ref[...] = p_ref[0] + p_ref[1]


_add2 = pl.pallas_call(
    _add2_body,
    out_shape=jax.ShapeDtypeStruct((N_PAD, D), jnp.float32),
)


def _final_body(e0_ref, e1_ref, e2_ref, p3_ref, o_ref):
    o_ref[...] = 0.25 * (e0_ref[...] + e1_ref[...] + e2_ref[...]
                         + p3_ref[0] + p3_ref[1])


_final = pl.pallas_call(
    _final_body,
    out_shape=jax.ShapeDtypeStruct((N_PAD, D), jnp.float32),
)


@jax.jit
def kernel(edge_index, edge_values, user_emb, item_emb, brand_emb):
    ego0 = jnp.concatenate(
        [user_emb, item_emb, brand_emb,
         jnp.zeros((N_PAD - N_NODES, D), jnp.float32)], axis=0)
    dst = edge_index[0]
    src = edge_index[1]

    pad = E_PAD - N_EDGES
    fill = (jnp.arange(pad, dtype=jnp.int32) * 97) % N_NODES
    src_p = jnp.concatenate([src, fill]).reshape(NW, CH, CW)
    dst_p = jnp.concatenate([dst, fill]).reshape(NW, CH, CW)
    val_p = jnp.concatenate(
        [edge_values, jnp.zeros((pad,), jnp.float32)]).reshape(NW, CH, CW)

    p1 = _layer_sc(ego0, src_p, dst_p, val_p)
    ego1 = _add2(p1)
    p2 = _layer_sc(ego1, src_p, dst_p, val_p)
    ego2 = _add2(p2)
    p3 = _layer_sc(ego2, src_p, dst_p, val_p)
    final = _final(ego0, ego1, ego2, p3)

    final_user = final[:NUM_USERS]
    final_item = final[NUM_USERS:NUM_USERS + NUM_ITEMS]
    return (final_user, final_item, user_emb, item_emb)
